# Initial kernel scaffold; baseline (speedup 1.0000x reference)
import jax
import jax.numpy as jnp
from jax.experimental import pallas as pl


def kernel(x, edge_index, edge_type, basis0, comp0, bias0, basis1, comp1, bias1):
    raise NotImplementedError("write your pallas kernel here")


# speedup vs baseline: 1.9258x; 1.9258x over previous
import functools

import jax
import jax.numpy as jnp
from jax import lax
from jax.experimental import pallas as pl
from jax.experimental.pallas import tpu as pltpu
from jax.experimental.pallas import tpu_sc as plsc

N_NODES = 10000
D = 128
NUM_REL = 32
NUM_BASES = 8
N_EDGES = 320000

NC = 2
NS = 16
NW = NC * NS

CH = 128
NCHUNK = 80
EPW = NCHUNK * CH
E_PAD = NW * EPW
NACC = 10240
RPT = NACC // NS
DUMMY_DST = NACC - 1

NBLK = 400
EBLK = 512


def _hw_body(comp_ref, basis_ref, h_ref, out_ref, w_scr):
  @pl.when(pl.program_id(0) == 0)
  def _():
    for r in range(NUM_REL):
      acc = comp_ref[r, 0] * basis_ref[0]
      for b in range(1, NUM_BASES):
        acc = acc + comp_ref[r, b] * basis_ref[b]
      w_scr[r] = acc

  h = h_ref[...]
  for r in range(NUM_REL):
    out_ref[r] = jnp.dot(h, w_scr[r], preferred_element_type=jnp.float32)


def _project(h, basis, comp):
  n_blocks = N_NODES // NBLK
  return pl.pallas_call(
      _hw_body,
      grid=(n_blocks,),
      in_specs=[
          pl.BlockSpec(memory_space=pltpu.SMEM),
          pl.BlockSpec((NUM_BASES, D, D), lambda i: (0, 0, 0)),
          pl.BlockSpec((NBLK, D), lambda i: (i, 0)),
      ],
      out_specs=pl.BlockSpec((NUM_REL, NBLK, D), lambda i: (0, i, 0)),
      out_shape=jax.ShapeDtypeStruct((NUM_REL, N_NODES, D), jnp.float32),
      scratch_shapes=[pltpu.VMEM((NUM_REL, D, D), jnp.float32)],
      compiler_params=pltpu.CompilerParams(
          dimension_semantics=("arbitrary",)),
  )(comp, basis, h)


def _agg_body(hw_ref, edges_ref, acc_out, idx_v, gidx_v, rows_v, zf_v,
              acc_sh, sem):
  cid = lax.axis_index("c")
  sid = lax.axis_index("s")
  wid = cid * NS + sid

  zv = jnp.zeros((16,), jnp.float32)
  for r in range(16):
    for c in range(D // 16):
      zf_v[r, pl.ds(c * 16, 16)] = zv

  for j in range(RPT // 16):
    pltpu.sync_copy(zf_v, acc_sh.at[pl.ds(sid * RPT + j * 16, 16)])
  plsc.subcore_barrier()

  def chunk(c, carry):
    pltpu.sync_copy(edges_ref.at[wid, c], idx_v)
    for i in range(CH // 16):
      s = idx_v[0, pl.ds(i * 16, 16)]
      e = idx_v[2, pl.ds(i * 16, 16)]
      gidx_v[pl.ds(i * 16, 16)] = e * N_NODES + s
    pltpu.async_copy(hw_ref.at[gidx_v], rows_v, sem).wait()
    pltpu.sync_copy(rows_v, acc_sh.at[idx_v.at[1]], add=True)
    return carry

  lax.fori_loop(0, NCHUNK, chunk, 0)
  plsc.subcore_barrier()

  pltpu.sync_copy(acc_sh.at[pl.ds(sid * RPT, RPT)],
                  acc_out.at[pl.ds(cid * NACC + sid * RPT, RPT)])


def _make_agg():
  mesh = plsc.VectorSubcoreMesh(
      core_axis_name="c", subcore_axis_name="s",
      num_cores=NC, num_subcores=NS)
  return pl.kernel(
      _agg_body,
      out_type=jax.ShapeDtypeStruct((NC * NACC, D), jnp.float32),
      mesh=mesh,
      scratch_types=[
          pltpu.VMEM((3, CH), jnp.int32),
          pltpu.VMEM((CH,), jnp.int32),
          pltpu.VMEM((CH, D), jnp.float32),
          pltpu.VMEM((16, D), jnp.float32),
          pltpu.VMEM_SHARED((NACC, D), jnp.float32),
          pltpu.SemaphoreType.DMA,
      ])


_sc_aggregate = _make_agg()


def _deg_body(edges_ref, deg_out, idx_v, ones_v, zf_v, deg_sh, sem):
  cid = lax.axis_index("c")
  sid = lax.axis_index("s")
  wid = cid * NS + sid

  zv = jnp.zeros((16,), jnp.float32)
  ov = jnp.ones((16,), jnp.float32)
  for r in range(16):
    for c in range(D // 16):
      zf_v[r, pl.ds(c * 16, 16)] = zv
  for r in range(CH):
    for c in range(D // 16):
      ones_v[r, pl.ds(c * 16, 16)] = ov

  for j in range(RPT // 16):
    pltpu.sync_copy(zf_v, deg_sh.at[pl.ds(sid * RPT + j * 16, 16)])
  plsc.subcore_barrier()

  def chunk(c, carry):
    pltpu.sync_copy(edges_ref.at[wid, c], idx_v)
    pltpu.sync_copy(ones_v, deg_sh.at[idx_v.at[1]], add=True)
    return carry

  lax.fori_loop(0, NCHUNK, chunk, 0)
  plsc.subcore_barrier()

  pltpu.sync_copy(deg_sh.at[pl.ds(sid * RPT, RPT)],
                  deg_out.at[pl.ds(cid * NACC + sid * RPT, RPT)])


def _make_deg():
  mesh = plsc.VectorSubcoreMesh(
      core_axis_name="c", subcore_axis_name="s",
      num_cores=NC, num_subcores=NS)
  return pl.kernel(
      _deg_body,
      out_type=jax.ShapeDtypeStruct((NC * NACC, D), jnp.float32),
      mesh=mesh,
      scratch_types=[
          pltpu.VMEM((3, CH), jnp.int32),
          pltpu.VMEM((CH, D), jnp.float32),
          pltpu.VMEM((16, D), jnp.float32),
          pltpu.VMEM_SHARED((NACC, D), jnp.float32),
          pltpu.SemaphoreType.DMA,
      ])


_sc_degree = _make_deg()


def _epi_body(bias_ref, p0_ref, p1_ref, d0_ref, d1_ref, out_ref):
  deg = d0_ref[...][:, :1] + d1_ref[...][:, :1]
  norm = 1.0 / jnp.maximum(deg, 1.0)
  agg = p0_ref[...] + p1_ref[...]
  out_ref[...] = jnp.maximum(agg * norm + bias_ref[...], 0.0)


def _epilogue(acc, deg, bias):
  n_blocks = NACC // EBLK
  return pl.pallas_call(
      _epi_body,
      grid=(n_blocks,),
      in_specs=[
          pl.BlockSpec((1, D), lambda i: (0, 0)),
          pl.BlockSpec((EBLK, D), lambda i: (i, 0)),
          pl.BlockSpec((EBLK, D), lambda i: (i + n_blocks, 0)),
          pl.BlockSpec((EBLK, D), lambda i: (i, 0)),
          pl.BlockSpec((EBLK, D), lambda i: (i + n_blocks, 0)),
      ],
      out_specs=pl.BlockSpec((EBLK, D), lambda i: (i, 0)),
      out_shape=jax.ShapeDtypeStruct((NACC, D), jnp.float32),
  )(bias.reshape(1, D), acc, acc, deg, deg)


@jax.jit
def kernel(x, edge_index, edge_type, basis0, comp0, bias0,
           basis1, comp1, bias1):
  src = edge_index[0].astype(jnp.int32)
  dst = edge_index[1].astype(jnp.int32)
  et = edge_type.astype(jnp.int32)

  pad = E_PAD - N_EDGES
  src = jnp.concatenate([src, jnp.zeros((pad,), jnp.int32)])
  dst = jnp.concatenate([dst, jnp.full((pad,), DUMMY_DST, jnp.int32)])
  et = jnp.concatenate([et, jnp.zeros((pad,), jnp.int32)])
  edges = jnp.stack([src, dst, et]).reshape(3, NW, NCHUNK, CH)
  edges = edges.transpose(1, 2, 0, 3)

  deg = _sc_degree(edges)

  hw1 = _project(x, basis0, comp0).reshape(NUM_REL * N_NODES, D)
  acc1 = _sc_aggregate(hw1, edges)
  h1 = _epilogue(acc1, deg, bias0)

  hw2 = _project(h1, basis1, comp1).reshape(NUM_REL * N_NODES, D)
  acc2 = _sc_aggregate(hw2, edges)
  out = _epilogue(acc2, deg, bias1)
  return out[:N_NODES]
